# baseline (speedup 1.0000x reference)
import jax
import jax.numpy as jnp
from jax.experimental import pallas as pl

C_IN = 26
C_OUT = 8
HID = 128
B = 1024

A_BLK = 128


def _stage_a_body(x_ref, w1t_ref, b1_ref, h_ref):
    w1t = w1t_ref[:]
    b1 = b1_ref[:]
    for c in range(C_IN):
        h = jnp.dot(x_ref[:, c, :], w1t, preferred_element_type=jnp.float32)
        h_ref[:, c, :] = jnp.maximum(h + b1, 0.0)


def _stage_bcd_body(f_ref, x_ref, wlt_ref, bl_ref, wrt_ref, out_ref):
    fi = f_ref[:]
    f = fi
    c_iota = jax.lax.broadcasted_iota(jnp.int32, (B, C_IN), 1).astype(jnp.float32)
    rank = jnp.zeros((B, C_IN), jnp.float32)
    for cp in range(C_IN):
        col = f[:, cp:cp + 1]
        gt = (col > f).astype(jnp.float32)
        tie = jnp.logical_and(col == f, cp < c_iota).astype(jnp.float32)
        rank = rank + gt + tie
    t = (rank < float(C_OUT)).astype(jnp.float32)
    pos = jnp.zeros((B, C_IN), jnp.float32)
    for cp in range(C_IN):
        pos = pos + t[:, cp:cp + 1] * (cp < c_iota).astype(jnp.float32)
    key = jnp.where(t > 0.0, pos, -1.0)
    g = jax.lax.dot_general(t, t, (((1,), (1,)), ((), ())),
                            preferred_element_type=jnp.float32)
    s = (g >= 1.5).astype(jnp.float32)
    deg = jnp.dot(s, t, preferred_element_type=jnp.float32)
    fit = fi * t

    out_ref[:] = jnp.zeros_like(out_ref)
    for c in range(C_IN):
        keyc = key[:, c:c + 1]
        degc = deg[:, c:c + 1]
        tc = (keyc >= 0.0).astype(jnp.float32)
        y = x_ref[:, c, :] * fit[:, c:c + 1]
        z = jnp.dot(s, y, preferred_element_type=jnp.float32)
        aggr = z / jnp.maximum(degc, 1.0) * tc
        nt = (((1,), (1,)), ((), ()))
        o = (jax.lax.dot_general(aggr, wlt_ref[c], nt,
                                 preferred_element_type=jnp.float32)
             + bl_ref[c]
             + jax.lax.dot_general(y, wrt_ref[c], nt,
                                   preferred_element_type=jnp.float32))
        o = jnp.maximum(o, 0.0)
        cnt = jnp.maximum(jnp.sum(tc) * float(HID), 1.0)
        mean = jnp.sum(o * tc) / cnt
        dv = o - mean
        var = jnp.sum(dv * dv * tc) / cnt
        onm = dv * jax.lax.rsqrt(var + 1e-5)
        for k in range(C_OUT):
            mk = (keyc == float(k)).astype(jnp.float32)
            out_ref[:, k, :] += onm * mk


def kernel(input_embedding, W1, b1, ln_g, ln_b, W2, b2, Wl, bl, Wr):
    x = input_embedding.astype(jnp.float32)

    h = pl.pallas_call(
        _stage_a_body,
        grid=(B // A_BLK,),
        in_specs=[
            pl.BlockSpec((A_BLK, C_IN, HID), lambda i: (i, 0, 0)),
            pl.BlockSpec((HID, HID), lambda i: (0, 0)),
            pl.BlockSpec((1, HID), lambda i: (0, 0)),
        ],
        out_specs=pl.BlockSpec((A_BLK, C_IN, HID), lambda i: (i, 0, 0)),
        out_shape=jax.ShapeDtypeStruct((B, C_IN, HID), jnp.float32),
    )(x, W1.T, b1.reshape(1, HID))

    mu = h.mean(-1, keepdims=True)
    var = h.var(-1, keepdims=True)
    hn = (h - mu) / jnp.sqrt(var + 1e-5) * ln_g + ln_b
    f = (jnp.einsum('bch,oh->bco', hn, W2) + b2)[..., 0]
    f = jax.nn.softmax(f, axis=1)

    return pl.pallas_call(
        _stage_bcd_body,
        out_shape=jax.ShapeDtypeStruct((B, C_OUT, HID), jnp.float32),
    )(f, x, Wl, bl.reshape(C_IN, 1, HID), Wr)


# speedup vs baseline: 1.3849x; 1.3849x over previous
import jax
import jax.numpy as jnp
from jax.experimental import pallas as pl

C_IN = 26
C_OUT = 8
HID = 128
B = 1024

A_BLK = 128


def _stage_a_body(x_ref, w1t_ref, b1_ref, h_ref):
    w1t = w1t_ref[:]
    b1 = b1_ref[:]
    for c in range(C_IN):
        h = jnp.dot(x_ref[:, c, :], w1t, preferred_element_type=jnp.float32)
        h_ref[:, c, :] = jnp.maximum(h + b1, 0.0)


def _stage_bcd_body(f_ref, x_ref, wlt_ref, bl_ref, wrt_ref, out_ref):
    fi = f_ref[:]
    f = fi
    c_iota = jax.lax.broadcasted_iota(jnp.int32, (B, C_IN), 1).astype(jnp.float32)
    rank = jnp.zeros((B, C_IN), jnp.float32)
    for cp in range(C_IN):
        col = f[:, cp:cp + 1]
        gt = (col > f).astype(jnp.float32)
        tie = jnp.logical_and(col == f, cp < c_iota).astype(jnp.float32)
        rank = rank + gt + tie
    t = (rank < float(C_OUT)).astype(jnp.float32)
    pos = jnp.zeros((B, C_IN), jnp.float32)
    for cp in range(C_IN):
        pos = pos + t[:, cp:cp + 1] * (cp < c_iota).astype(jnp.float32)
    key = jnp.where(t > 0.0, pos, -1.0)
    g = jax.lax.dot_general(t, t, (((1,), (1,)), ((), ())),
                            preferred_element_type=jnp.float32)
    s = (g >= 1.5).astype(jnp.float32)
    deg = jnp.dot(s, t, preferred_element_type=jnp.float32)
    fit = fi * t

    out_ref[:] = jnp.zeros_like(out_ref)
    for c in range(C_IN):
        keyc = key[:, c:c + 1]
        degc = deg[:, c:c + 1]
        tc = (keyc >= 0.0).astype(jnp.float32)
        y = x_ref[:, c, :] * fit[:, c:c + 1]
        z = jnp.dot(s, y, preferred_element_type=jnp.float32)
        aggr = z / jnp.maximum(degc, 1.0) * tc
        o = (jnp.dot(aggr, wlt_ref[c], preferred_element_type=jnp.float32)
             + bl_ref[c]
             + jnp.dot(y, wrt_ref[c], preferred_element_type=jnp.float32))
        o = jnp.maximum(o, 0.0)
        cnt = jnp.maximum(jnp.sum(tc) * float(HID), 1.0)
        mean = jnp.sum(o * tc) / cnt
        dv = o - mean
        var = jnp.sum(dv * dv * tc) / cnt
        onm = dv * jax.lax.rsqrt(var + 1e-5)
        for k in range(C_OUT):
            mk = (keyc == float(k)).astype(jnp.float32)
            out_ref[:, k * HID:(k + 1) * HID] += onm * mk


def kernel(input_embedding, W1, b1, ln_g, ln_b, W2, b2, Wl, bl, Wr):
    x = input_embedding.astype(jnp.float32)

    h = pl.pallas_call(
        _stage_a_body,
        grid=(B // A_BLK,),
        in_specs=[
            pl.BlockSpec((A_BLK, C_IN, HID), lambda i: (i, 0, 0)),
            pl.BlockSpec((HID, HID), lambda i: (0, 0)),
            pl.BlockSpec((1, HID), lambda i: (0, 0)),
        ],
        out_specs=pl.BlockSpec((A_BLK, C_IN, HID), lambda i: (i, 0, 0)),
        out_shape=jax.ShapeDtypeStruct((B, C_IN, HID), jnp.float32),
    )(x, W1.T, b1.reshape(1, HID))

    mu = h.mean(-1, keepdims=True)
    var = h.var(-1, keepdims=True)
    hn = (h - mu) / jnp.sqrt(var + 1e-5) * ln_g + ln_b
    f = (jnp.einsum('bch,oh->bco', hn, W2) + b2)[..., 0]
    f = jax.nn.softmax(f, axis=1)

    out2d = pl.pallas_call(
        _stage_bcd_body,
        out_shape=jax.ShapeDtypeStruct((B, C_OUT * HID), jnp.float32),
    )(f, x, jnp.transpose(Wl, (0, 2, 1)), bl.reshape(C_IN, 1, HID),
      jnp.transpose(Wr, (0, 2, 1)))

    return out2d.reshape(B, C_OUT, HID)
